# Initial kernel scaffold; baseline (speedup 1.0000x reference)
import functools

import jax
import jax.numpy as jnp
from jax import lax
from jax.experimental import pallas as pl
from jax.experimental.pallas import tpu as pltpu
from jax.experimental.pallas import tpu_sc as plsc

N = 10000
E = 320000
D = 128

NC = 2
NS = 16
NW = NC * NS
E_PER_TILE = E // NW
K = 80
CHUNKS = E_PER_TILE // K
ROWS_PER_TILE = N // NS


def _sc_segment_sum(x, src, dst, z128, z16, ones, with_deg):
    out_types = [jax.ShapeDtypeStruct((NC, N, D), jnp.float32)]
    if with_deg:
        out_types.append(jax.ShapeDtypeStruct((NC, N, 16), jnp.float32))

    scratch = [
        pltpu.VMEM((K,), jnp.int32),
        pltpu.VMEM((K,), jnp.int32),
        pltpu.VMEM((K, D), jnp.float32),
        pltpu.VMEM((K, 16), jnp.float32),
        pltpu.VMEM_SHARED((N, D), jnp.float32),
        pltpu.VMEM_SHARED((N, 16), jnp.float32),
        pltpu.SemaphoreType.DMA,
    ]

    def body(x_hbm, src_hbm, dst_hbm, z128_hbm, z16_hbm, ones_hbm,
             *refs):
        if with_deg:
            parts_hbm, degp_hbm = refs[0], refs[1]
            rest = refs[2:]
        else:
            parts_hbm = refs[0]
            rest = refs[1:]
        src_v, dst_v, rows_v, ones_v, acc_sh, deg_sh, sem = rest

        c = lax.axis_index("c")
        s = lax.axis_index("s")
        wid = c * NS + s
        rbase = s * ROWS_PER_TILE

        pltpu.sync_copy(z128_hbm.at[pl.ds(rbase, ROWS_PER_TILE)],
                        acc_sh.at[pl.ds(rbase, ROWS_PER_TILE)])
        if with_deg:
            pltpu.sync_copy(z16_hbm.at[pl.ds(rbase, ROWS_PER_TILE)],
                            deg_sh.at[pl.ds(rbase, ROWS_PER_TILE)])
            pltpu.sync_copy(ones_hbm, ones_v)
        plsc.subcore_barrier()

        @pl.loop(0, CHUNKS)
        def _(i):
            ebase = wid * E_PER_TILE + i * K
            pltpu.sync_copy(src_hbm.at[pl.ds(ebase, K)], src_v)
            pltpu.sync_copy(dst_hbm.at[pl.ds(ebase, K)], dst_v)
            pltpu.async_copy(x_hbm.at[src_v], rows_v, sem).wait()
            pltpu.sync_copy(rows_v, acc_sh.at[dst_v], add=True)
            if with_deg:
                pltpu.sync_copy(ones_v, deg_sh.at[dst_v], add=True)

        plsc.subcore_barrier()

        pltpu.sync_copy(acc_sh.at[pl.ds(rbase, ROWS_PER_TILE)],
                        parts_hbm.at[c].at[pl.ds(rbase, ROWS_PER_TILE)])
        if with_deg:
            pltpu.sync_copy(deg_sh.at[pl.ds(rbase, ROWS_PER_TILE)],
                            degp_hbm.at[c].at[pl.ds(rbase, ROWS_PER_TILE)])

    mesh = plsc.VectorSubcoreMesh(core_axis_name="c", subcore_axis_name="s")
    kfn = pl.kernel(body, out_type=out_types, mesh=mesh,
                    scratch_types=scratch)
    return kfn(x, src, dst, z128, z16, ones)


def _tc_body(parts_ref, degp_ref, x_ref, wl_ref, bl_ref, wr_ref, o_ref, *,
             relu):
    p = parts_ref[0] + parts_ref[1]
    dgp = degp_ref[...]
    dg = dgp[0, :, 0] + dgp[1, :, 0]
    inv = 1.0 / jnp.maximum(dg, 1.0)
    agg = p * inv[:, None]
    acc = lax.dot_general(agg, wl_ref[...], (((1,), (1,)), ((), ())),
                          preferred_element_type=jnp.float32)
    acc = acc + bl_ref[...]
    acc = acc + lax.dot_general(x_ref[...], wr_ref[...],
                                (((1,), (1,)), ((), ())),
                                preferred_element_type=jnp.float32)
    o_ref[...] = jnp.maximum(acc, 0.0) if relu else acc


def _tc_combine(parts, degp, x, Wl, bl, Wr, relu):
    R = 2000
    grid = (N // R,)
    return pl.pallas_call(
        functools.partial(_tc_body, relu=relu),
        grid=grid,
        in_specs=[
            pl.BlockSpec((NC, R, D), lambda i: (0, i, 0)),
            pl.BlockSpec((NC, R, 16), lambda i: (0, i, 0)),
            pl.BlockSpec((R, D), lambda i: (i, 0)),
            pl.BlockSpec((D, D), lambda i: (0, 0)),
            pl.BlockSpec((1, D), lambda i: (0, 0)),
            pl.BlockSpec((D, D), lambda i: (0, 0)),
        ],
        out_specs=pl.BlockSpec((R, D), lambda i: (i, 0)),
        out_shape=jax.ShapeDtypeStruct((N, D), jnp.float32),
    )(parts, degp, x, Wl, bl.reshape(1, D), Wr)


def kernel(x, edge_index, W1l, b1l, W1r, W2l, b2l, W2r):
    src = edge_index[0].astype(jnp.int32)
    dst = edge_index[1].astype(jnp.int32)
    z128 = jnp.zeros((N, D), jnp.float32)
    z16 = jnp.zeros((N, 16), jnp.float32)
    ones = jnp.ones((K, 16), jnp.float32)

    parts1, degp = _sc_segment_sum(x, src, dst, z128, z16, ones, True)
    h = _tc_combine(parts1, degp, x, W1l, b1l, W1r, relu=True)
    (parts2,) = _sc_segment_sum(h, src, dst, z128, z16, ones, False)
    out = _tc_combine(parts2, degp, h, W2l, b2l, W2r, relu=False)
    return out


# speedup vs baseline: 4.3367x; 4.3367x over previous
import functools

import jax
import jax.numpy as jnp
from jax import lax
from jax.experimental import pallas as pl
from jax.experimental.pallas import tpu as pltpu
from jax.experimental.pallas import tpu_sc as plsc

N = 10000
NP_ = 10240
E = 320000
D = 128

NC = 2
NS = 16
NW = NC * NS
E_PER_TILE = E // NW
K = 80
CHUNKS = E_PER_TILE // K
ROWS_PER_TILE = NP_ // NS


def _sc_segment_sum(x, src, dst, z128, ones, with_deg):
    out_types = [jax.ShapeDtypeStruct((NC, NP_, D), jnp.float32)]
    if with_deg:
        out_types.append(jax.ShapeDtypeStruct((NC, NP_, D), jnp.float32))

    scratch = [
        pltpu.VMEM((K,), jnp.int32),
        pltpu.VMEM((K,), jnp.int32),
        pltpu.VMEM((K, D), jnp.float32),
        pltpu.VMEM((K, D), jnp.float32),
        pltpu.VMEM_SHARED((NP_, D), jnp.float32),
        pltpu.SemaphoreType.DMA,
    ]

    def body(x_hbm, src_hbm, dst_hbm, z128_hbm, ones_hbm, *refs):
        if with_deg:
            parts_hbm, degp_hbm = refs[0], refs[1]
            rest = refs[2:]
        else:
            parts_hbm = refs[0]
            rest = refs[1:]
        src_v, dst_v, rows_v, ones_v, acc_sh, sem = rest

        c = lax.axis_index("c")
        s = lax.axis_index("s")
        wid = c * NS + s
        rbase = s * ROWS_PER_TILE

        def zero_acc():
            @pl.loop(0, ROWS_PER_TILE // K)
            def _(j):
                rb = rbase + j * K
                pltpu.sync_copy(z128_hbm.at[pl.ds(rb, K)], rows_v)
                pltpu.sync_copy(rows_v, acc_sh.at[pl.ds(rb, K)])

        def copy_acc_out(dst_ref):
            @pl.loop(0, ROWS_PER_TILE // K)
            def _(j):
                rb = rbase + j * K
                pltpu.sync_copy(acc_sh.at[pl.ds(rb, K)], rows_v)
                pltpu.sync_copy(rows_v, dst_ref.at[c].at[pl.ds(rb, K)])

        zero_acc()
        plsc.subcore_barrier()

        @pl.loop(0, CHUNKS)
        def _(i):
            ebase = wid * E_PER_TILE + i * K
            pltpu.sync_copy(src_hbm.at[pl.ds(ebase, K)], src_v)
            pltpu.sync_copy(dst_hbm.at[pl.ds(ebase, K)], dst_v)
            pltpu.async_copy(x_hbm.at[src_v], rows_v, sem).wait()
            pltpu.sync_copy(rows_v, acc_sh.at[dst_v], add=True)

        plsc.subcore_barrier()
        copy_acc_out(parts_hbm)

        if with_deg:
            plsc.subcore_barrier()
            zero_acc()
            pltpu.sync_copy(ones_hbm, ones_v)
            plsc.subcore_barrier()

            @pl.loop(0, CHUNKS)
            def _(i):
                ebase = wid * E_PER_TILE + i * K
                pltpu.sync_copy(dst_hbm.at[pl.ds(ebase, K)], dst_v)
                pltpu.sync_copy(ones_v, acc_sh.at[dst_v], add=True)

            plsc.subcore_barrier()
            copy_acc_out(degp_hbm)

    mesh = plsc.VectorSubcoreMesh(core_axis_name="c", subcore_axis_name="s")
    kfn = pl.kernel(body, out_type=out_types, mesh=mesh,
                    scratch_types=scratch)
    return kfn(x, src, dst, z128, ones)


def _tc_body(parts_ref, degp_ref, x_ref, wl_ref, bl_ref, wr_ref, o_ref, *,
             relu, block_rows):
    p = parts_ref[0] + parts_ref[1]
    dg = degp_ref[0, :, :1] + degp_ref[1, :, :1]
    inv = 1.0 / jnp.maximum(dg, 1.0)
    agg = p * inv
    acc = lax.dot_general(agg, wl_ref[...], (((1,), (1,)), ((), ())),
                          preferred_element_type=jnp.float32)
    acc = acc + bl_ref[...]
    acc = acc + lax.dot_general(x_ref[...], wr_ref[...],
                                (((1,), (1,)), ((), ())),
                                preferred_element_type=jnp.float32)
    o_ref[...] = jnp.maximum(acc, 0.0) if relu else acc


def _tc_combine(parts, degp, x, Wl, bl, Wr, relu):
    R = 2000
    grid = (N // R,)
    return pl.pallas_call(
        functools.partial(_tc_body, relu=relu, block_rows=R),
        grid=grid,
        in_specs=[
            pl.BlockSpec((NC, R, D), lambda i: (0, i, 0)),
            pl.BlockSpec((NC, R, D), lambda i: (0, i, 0)),
            pl.BlockSpec((R, D), lambda i: (i, 0)),
            pl.BlockSpec((D, D), lambda i: (0, 0)),
            pl.BlockSpec((1, D), lambda i: (0, 0)),
            pl.BlockSpec((D, D), lambda i: (0, 0)),
        ],
        out_specs=pl.BlockSpec((R, D), lambda i: (i, 0)),
        out_shape=jax.ShapeDtypeStruct((N, D), jnp.float32),
    )(parts, degp, x, Wl, bl.reshape(1, D), Wr)


def kernel(x, edge_index, W1l, b1l, W1r, W2l, b2l, W2r):
    src = edge_index[0].astype(jnp.int32)
    dst = edge_index[1].astype(jnp.int32)
    z128 = jnp.zeros((NP_, D), jnp.float32)
    ones = jnp.ones((K, D), jnp.float32)

    parts1, degp = _sc_segment_sum(x, src, dst, z128, ones, True)
    h = _tc_combine(parts1, degp, x, W1l, b1l, W1r, relu=True)
    (parts2,) = _sc_segment_sum(h, src, dst, z128, ones, False)
    out = _tc_combine(parts2, degp, h, W2l, b2l, W2r, relu=False)
    return out
